# baseline (speedup 1.0000x reference)
import jax
import jax.numpy as jnp
from jax import lax
from jax.experimental import pallas as pl
from jax.experimental.pallas import tpu as pltpu
from jax.experimental.pallas import tpu_sc as plsc

N = 10000
D = 128
E = 320000

NC = 2
NS = 16
NW = NC * NS

EPW = E // NW
K = 40
NCHUNK = EPW // K
B = 5
NG = 5
PG = 3
NR = 10
U = 10
OUTER = NCHUNK // U
RPT = 624
TAIL = N - NS * RPT


def _sc_spmm_body(x_hbm, c0_h, r0_h, v0_h, c1_h, r1_h, v1_h, out_hbm,
                  acc, colv, rowv, valv, gbuf, *sems):
    isem = sems[:2]
    rsem = sems[2:2 + NR]
    gsem = sems[2 + NR:2 + NR + NG]
    asem = sems[2 + NR + NG:]
    cid = lax.axis_index("c")
    sid = lax.axis_index("s")
    wid = sid * NC + cid
    wbase = wid * EPW

    def coff(u):
        return ((u // B) % 2) * B * K + (u % B) * K

    def start_gather(u, b):
        pltpu.async_copy(x_hbm.at[colv.at[pl.ds(coff(u), K)]],
                         gbuf.at[b], gsem[b])

    def wait_gather(u, b):
        pltpu.make_async_copy(x_hbm.at[colv.at[pl.ds(coff(u), K)]],
                              gbuf.at[b], gsem[b]).wait()

    def wait_scatter(s, b):
        pltpu.make_async_copy(gbuf.at[b], acc.at[rowv.at[s]],
                              asem[b]).wait()

    def run_pass(cols_h, rows_h, vals_h):
        def fill(m, hb):
            cs = pl.ds(wbase + m * B * K, B * K)
            sb = pl.ds(hb * B * K, B * K)
            pltpu.async_copy(cols_h.at[cs], colv.at[sb], isem[hb])
            pltpu.async_copy(vals_h.at[cs], valv.at[sb], isem[hb])

        def wait_fill(hb):
            sb = pl.ds(hb * B * K, B * K)
            pltpu.make_async_copy(cols_h.at[pl.ds(0, B * K)], colv.at[sb],
                                  isem[hb]).wait()
            pltpu.make_async_copy(vals_h.at[pl.ds(0, B * K)], valv.at[sb],
                                  isem[hb]).wait()

        def start_rows(gq, s):
            pltpu.async_copy(rows_h.at[pl.ds(wbase + gq * K, K)],
                             rowv.at[s], rsem[s])

        def wait_rows(s):
            pltpu.make_async_copy(rows_h.at[pl.ds(0, K)], rowv.at[s],
                                  rsem[s]).wait()

        fill(0, 0)
        for j in range(PG + 1):
            start_rows(j, j)
        wait_fill(0)
        for j in range(PG):
            start_gather(j, j)

        def _outer(o, _):
            for u in range(U):
                g = o * U + u
                b = u % NG
                s = u % NR
                s2 = (u - 2) % NR
                bb = (u - 2) % NG
                sp = (u + 4) % NR

                if u <= 1:
                    pl.when(o > 0)(lambda: wait_scatter(s2, bb))
                else:
                    wait_scatter(s2, bb)

                if u == 1:
                    fill(2 * o + 1, 1)
                elif u == 6:
                    pl.when(o < OUTER - 1)(lambda: fill(2 * o + 2, 0))

                def _rows():
                    start_rows(g + 4, sp)
                if u >= U - 4:
                    pl.when(o < OUTER - 1)(_rows)
                else:
                    _rows()

                def _next_gather():
                    if u == 2:
                        wait_fill(1)
                    elif u == 7:
                        wait_fill(0)
                    start_gather(u + PG, bb)
                if u >= U - PG:
                    pl.when(o < OUTER - 1)(_next_gather)
                else:
                    _next_gather()

                wait_gather(u, b)
                gb = gbuf.at[b]
                vo = coff(u)

                @plsc.parallel_loop(0, K, step=1, unroll=4)
                def _scale(e):
                    vbc = plsc.load_gather(
                        valv, [jnp.full((16,), vo, jnp.int32) + e])
                    for d in range(D // 16):
                        sl = pl.ds(d * 16, 16)
                        gb[e, sl] = gb[e, sl] * vbc

                wait_rows(s)
                pltpu.async_copy(gb, acc.at[rowv.at[s]], asem[b], add=True)
            return _

        lax.fori_loop(0, OUTER, _outer, None)

        wait_scatter((U - 2) % NR, (U - 2) % NG)
        wait_scatter((U - 1) % NR, (U - 1) % NG)

    def zero_acc():
        zeros = jnp.zeros((16,), jnp.float32)

        @plsc.parallel_loop(0, K, step=1, unroll=4)
        def _zrow(r):
            for d in range(D // 16):
                gbuf[NG - 1, r, pl.ds(d * 16, 16)] = zeros

        zsrc = gbuf.at[NG - 1]
        for j in range(RPT // K):
            pltpu.sync_copy(zsrc, acc.at[pl.ds(sid * RPT + j * K, K)])
        rem = RPT % K
        if rem:
            pltpu.sync_copy(zsrc.at[pl.ds(0, rem)],
                            acc.at[pl.ds(sid * RPT + (RPT // K) * K, rem)])

        @pl.when(sid == 0)
        def _ztail():
            pltpu.sync_copy(zsrc.at[pl.ds(0, TAIL)],
                            acc.at[pl.ds(NS * RPT, TAIL)])

    zero_acc()
    plsc.subcore_barrier()

    run_pass(c0_h, r0_h, v0_h)
    run_pass(c1_h, r1_h, v1_h)

    plsc.subcore_barrier()

    pltpu.sync_copy(acc.at[pl.ds(sid * RPT, RPT)],
                    out_hbm.at[cid, pl.ds(sid * RPT, RPT)])

    @pl.when(sid == 0)
    def _dtail():
        pltpu.sync_copy(acc.at[pl.ds(NS * RPT, TAIL)],
                        out_hbm.at[cid, pl.ds(NS * RPT, TAIL)])


def _sc_spmm(x, c0, r0, v0, c1, r1, v1):
    mesh = plsc.VectorSubcoreMesh(core_axis_name="c", subcore_axis_name="s")
    f = pl.kernel(
        _sc_spmm_body,
        out_type=jax.ShapeDtypeStruct((NC, N, D), jnp.float32),
        mesh=mesh,
        scratch_types=[
            pltpu.VMEM_SHARED((N, D), jnp.float32),
            pltpu.VMEM((2 * B * K,), jnp.int32),
            pltpu.VMEM((NR, K), jnp.int32),
            pltpu.VMEM((2 * B * K,), jnp.float32),
            pltpu.VMEM((NG, K, D), jnp.float32),
        ] + [pltpu.SemaphoreType.DMA] * (2 + NR + 2 * NG),
        compiler_params=pltpu.CompilerParams(needs_layout_passes=False),
    )
    return f(x, c0, r0, v0, c1, r1, v1)


def _mm_body(p_ref, w_ref, b_ref, o_ref):
    xblk = p_ref[0] + p_ref[1]
    o_ref[...] = (
        jnp.dot(xblk, w_ref[...], preferred_element_type=jnp.float32)
        + b_ref[...]
    )


def _mm(p, weight, bias):
    mb = 1000
    grid = (N // mb,)
    return pl.pallas_call(
        _mm_body,
        grid=grid,
        in_specs=[
            pl.BlockSpec((NC, mb, D), lambda i: (0, i, 0)),
            pl.BlockSpec((D, D), lambda i: (0, 0)),
            pl.BlockSpec((1, D), lambda i: (0, 0)),
        ],
        out_specs=pl.BlockSpec((mb, D), lambda i: (i, 0)),
        out_shape=jax.ShapeDtypeStruct((N, D), jnp.float32),
    )(p, weight, bias)


@jax.jit
def kernel(input, weight, bias, vals0, vals1, rows0, cols0, rows1, cols1):
    p = _sc_spmm(input, cols0, rows0, vals0, cols1, rows1, vals1)
    return _mm(p, weight, bias.reshape(1, D))
